# baseline (speedup 1.0000x reference)
import functools

import jax
import jax.numpy as jnp
from jax import lax
from jax.experimental import pallas as pl
from jax.experimental.pallas import tpu as pltpu
from jax.experimental.pallas import tpu_sc as plsc

NUM_FEATURES = 512
ZMAX = 87
CONFIG_DIM = 20
N_ATOMS = 100000

NC = 2
NS = 16
NW = NC * NS

LANES = 16
NSLICE = NUM_FEATURES // LANES

CHUNK = 80
NCH = N_ATOMS // CHUNK
T_MAX = (NCH + NW - 1) // NW
S_MAX = (T_MAX + 1) // 2


def _table_body(ee_ref, ec_ref, w_ref, out_ref):
    out_ref[...] = ee_ref[...] + lax.dot_general(
        ec_ref[...], w_ref[...],
        dimension_numbers=(((1,), (1,)), ((), ())),
        preferred_element_type=jnp.float32,
    )


def _build_table(element_embedding, electron_config, W):
    return pl.pallas_call(
        _table_body,
        out_shape=jax.ShapeDtypeStruct((ZMAX, NUM_FEATURES), jnp.float32),
    )(element_embedding, electron_config, W)


_mesh = plsc.VectorSubcoreMesh(
    core_axis_name="c", subcore_axis_name="s", num_cores=NC, num_subcores=NS
)


@functools.partial(
    pl.kernel,
    out_type=jax.ShapeDtypeStruct((N_ATOMS, NUM_FEATURES), jnp.float32),
    mesh=_mesh,
    scratch_types=[
        pltpu.VMEM((ZMAX, NUM_FEATURES), jnp.float32),
        pltpu.VMEM((CHUNK, NUM_FEATURES), jnp.float32),
        pltpu.VMEM((CHUNK, NUM_FEATURES), jnp.float32),
        pltpu.VMEM((2, CHUNK + LANES), jnp.int32),
        pltpu.SemaphoreType.DMA,
        pltpu.SemaphoreType.DMA,
        pltpu.SemaphoreType.DMA,
        pltpu.SemaphoreType.DMA,
        pltpu.SemaphoreType.DMA,
    ],
)
def _gather_kernel(table_hbm, z_hbm, out_hbm, table_v, stag0, stag1, idx_v,
                   so0, so1, si0, si1, st):
    wid = lax.axis_index("s") * NC + lax.axis_index("c")
    stag = (stag0, stag1)
    so = (so0, so1)
    si = (si0, si1)

    table_copy = pltpu.async_copy(table_hbm, table_v, st)

    def cid_of(tt):
        return wid + tt * NW

    def out_wait(b):
        pltpu.make_async_copy(
            stag[b], out_hbm.at[pl.ds(0, CHUNK)], so[b]
        ).wait()

    def idx_start(tt, b):
        @pl.when(cid_of(tt) < NCH)
        def _():
            pltpu.async_copy(
                z_hbm.at[pl.ds(cid_of(tt) * CHUNK, CHUNK)],
                idx_v.at[b, pl.ds(0, CHUNK)],
                si[b],
            )

    def idx_wait(b):
        pltpu.make_async_copy(
            z_hbm.at[pl.ds(0, CHUNK)], idx_v.at[b, pl.ds(0, CHUNK)], si[b]
        ).wait()

    idx_start(0, 0)
    idx_start(1, 1)
    table_copy.wait()

    def do_chunk(tt, b, first):
        cid = cid_of(tt)

        @pl.when(cid < NCH)
        def _():
            base = cid * CHUNK
            idx_wait(b)

            @pl.when(jnp.logical_not(first))
            def _():
                out_wait(b)

            @plsc.parallel_loop(0, CHUNK // 8, unroll=2)
            def _(g):
                zv = idx_v[b, pl.ds(g * 8, LANES)]
                for j in range(8):
                    z = zv[j]
                    i = g * 8 + j
                    for f in range(NSLICE):
                        stag[b][i, pl.ds(f * LANES, LANES)] = (
                            table_v[z, pl.ds(f * LANES, LANES)]
                        )
            idx_start(tt + 2, b)
            pltpu.async_copy(stag[b], out_hbm.at[pl.ds(base, CHUNK)], so[b])

    def body(s, carry):
        for b in (0, 1):
            do_chunk(2 * s + b, b, s == 0)
        return carry

    lax.fori_loop(0, S_MAX, body, 0)

    out_wait(0)
    out_wait(1)


def kernel(Z, element_embedding, W, electron_config):
    table = _build_table(element_embedding, electron_config, W)
    return _gather_kernel(table, Z.astype(jnp.int32))


# speedup vs baseline: 4.0634x; 4.0634x over previous
import functools

import jax
import jax.numpy as jnp
from jax import lax
from jax.experimental import pallas as pl
from jax.experimental.pallas import tpu as pltpu
from jax.experimental.pallas import tpu_sc as plsc

NUM_FEATURES = 512
ZMAX = 87
CONFIG_DIM = 20
N_ATOMS = 100000

NC = 2
NS = 16
NW = NC * NS

LANES = 16
NSLICE = NUM_FEATURES // LANES

CHUNK = 80
NCH = N_ATOMS // CHUNK
T_MAX = (NCH + NW - 1) // NW
S_MAX = (T_MAX + 1) // 2


def _table_body(ee_ref, ec_ref, w_ref, out_ref):
    out_ref[...] = ee_ref[...] + lax.dot_general(
        ec_ref[...], w_ref[...],
        dimension_numbers=(((1,), (1,)), ((), ())),
        preferred_element_type=jnp.float32,
    )


def _build_table(element_embedding, electron_config, W):
    return pl.pallas_call(
        _table_body,
        out_shape=jax.ShapeDtypeStruct((ZMAX, NUM_FEATURES), jnp.float32),
    )(element_embedding, electron_config, W)


_mesh = plsc.VectorSubcoreMesh(
    core_axis_name="c", subcore_axis_name="s", num_cores=NC, num_subcores=NS
)


@functools.partial(
    pl.kernel,
    out_type=jax.ShapeDtypeStruct((N_ATOMS, NUM_FEATURES), jnp.float32),
    mesh=_mesh,
    scratch_types=[
        pltpu.VMEM((ZMAX, NUM_FEATURES), jnp.float32),
        pltpu.VMEM((CHUNK, NUM_FEATURES), jnp.float32),
        pltpu.VMEM((CHUNK, NUM_FEATURES), jnp.float32),
        pltpu.VMEM((2, CHUNK + LANES), jnp.int32),
        pltpu.SemaphoreType.DMA,
        pltpu.SemaphoreType.DMA,
        pltpu.SemaphoreType.DMA,
        pltpu.SemaphoreType.DMA,
        pltpu.SemaphoreType.DMA,
    ],
)
def _gather_kernel(table_hbm, z_hbm, out_hbm, table_v, stag0, stag1, idx_v,
                   so0, so1, si0, si1, st):
    wid = lax.axis_index("s") * NC + lax.axis_index("c")
    stag = (stag0, stag1)
    so = (so0, so1)
    si = (si0, si1)

    table_copy = pltpu.async_copy(table_hbm, table_v, st)

    def cid_of(tt):
        return wid + tt * NW

    def out_wait(b):
        pltpu.make_async_copy(
            stag[b], out_hbm.at[pl.ds(0, CHUNK)], so[b]
        ).wait()

    def idx_start(tt, b):
        @pl.when(cid_of(tt) < NCH)
        def _():
            pltpu.async_copy(
                z_hbm.at[pl.ds(cid_of(tt) * CHUNK, CHUNK)],
                idx_v.at[b, pl.ds(0, CHUNK)],
                si[b],
            )

    def idx_wait(b):
        pltpu.make_async_copy(
            z_hbm.at[pl.ds(0, CHUNK)], idx_v.at[b, pl.ds(0, CHUNK)], si[b]
        ).wait()

    idx_start(0, 0)
    idx_start(1, 1)
    table_copy.wait()

    def do_chunk(tt, b, first):
        cid = cid_of(tt)

        @pl.when(cid < NCH)
        def _():
            base = cid * CHUNK
            idx_wait(b)

            @pl.when(jnp.logical_not(first))
            def _():
                out_wait(b)

            @plsc.parallel_loop(0, CHUNK, unroll=8)
            def _(i):
                z = idx_v[b, pl.ds(i, LANES)][0]
                for f in range(NSLICE):
                    stag[b][i, pl.ds(f * LANES, LANES)] = (
                        table_v[z, pl.ds(f * LANES, LANES)]
                    )
            idx_start(tt + 2, b)
            pltpu.async_copy(stag[b], out_hbm.at[pl.ds(base, CHUNK)], so[b])

    def body(s, carry):
        for b in (0, 1):
            do_chunk(2 * s + b, b, s == 0)
        return carry

    lax.fori_loop(0, S_MAX, body, 0)

    out_wait(0)
    out_wait(1)


def kernel(Z, element_embedding, W, electron_config):
    table = _build_table(element_embedding, electron_config, W)
    return _gather_kernel(table, Z.astype(jnp.int32))
